# Initial kernel scaffold; baseline (speedup 1.0000x reference)
import jax
import jax.numpy as jnp
from jax.experimental import pallas as pl


def kernel(x, embedding, pos_encoding):
    raise NotImplementedError("write your pallas kernel here")


# speedup vs baseline: 3.9144x; 3.9144x over previous
import functools

import jax
import jax.numpy as jnp
from jax import lax
from jax.experimental import pallas as pl
from jax.experimental.pallas import tpu as pltpu
from jax.experimental.pallas import tpu_sc as plsc

NC = 2
NS = 16
NW = NC * NS
LANES = 16

NBUF = 4


def _make_kernel(B, S, D, V):
    N = B * S
    seq_per_w = (N // S) // NW
    half = S // 2
    rounds = seq_per_w // NBUF

    mesh = plsc.VectorSubcoreMesh(core_axis_name="c", subcore_axis_name="s")

    @functools.partial(
        pl.kernel,
        out_type=jax.ShapeDtypeStruct((N, D), jnp.float32),
        mesh=mesh,
        compiler_params=pltpu.CompilerParams(use_tc_tiling_on_sc=False),
        scratch_types=[
            pltpu.VMEM((S, D), jnp.float32),
            pltpu.VMEM((NBUF, 2, half), jnp.int32),
            pltpu.VMEM((NBUF, S, D), jnp.float32),
            pltpu.SemaphoreType.DMA((NBUF,)),
            pltpu.SemaphoreType.DMA((NBUF,)),
            pltpu.SemaphoreType.DMA((NBUF,)),
        ],
    )
    def emb_kernel(idx_hbm, pos_hbm, table_hbm, out_hbm,
                   pos_v, idx_v, rows_v, si, sg, so):
        cid = lax.axis_index("c")
        sid = lax.axis_index("s")
        wid = sid * NC + cid
        base_seq = wid * seq_per_w

        pltpu.sync_copy(pos_hbm, pos_v)

        def idx_copy(seq, b):
            return pltpu.make_async_copy(
                idx_hbm.at[pl.ds(seq * 2, 2)], idx_v.at[b], si.at[b])

        def gather_copy(seq, b, j):
            return pltpu.make_async_copy(
                table_hbm.at[idx_v.at[b, j]],
                rows_v.at[b, pl.ds(j * half, half)],
                sg.at[b])

        def out_copy(seq, b):
            return pltpu.make_async_copy(
                rows_v.at[b], out_hbm.at[pl.ds(seq * S, S)], so.at[b])

        def add_pos(b):
            def body(i, carry):
                r = 2 * i
                for rr in (0, 1):
                    for c4 in range(D // LANES):
                        sl = pl.ds(c4 * LANES, LANES)
                        plsc.addupdate(rows_v.at[b, r + rr, sl],
                                       pos_v[r + rr, sl])
                return carry
            lax.fori_loop(0, S // 2, body, 0, unroll=False)

        def fire(o, b, first):
            seq = base_seq + o * NBUF + b
            idx_copy(seq, b).wait()
            if not first:
                out_copy(seq - NBUF, b).wait()
            gather_copy(seq, b, 0).start()
            gather_copy(seq, b, 1).start()

        def compute(o, b, last):
            seq = base_seq + o * NBUF + b
            gather_copy(seq, b, 0).wait()
            gather_copy(seq, b, 1).wait()
            if not last:
                idx_copy(seq + NBUF, b).start()
            add_pos(b)
            out_copy(seq, b).start()

        for b in range(NBUF):
            idx_copy(base_seq + b, b).start()

        for b in range(NBUF):
            fire(0, b, first=True)
        for b in range(NBUF):
            compute(0, b, last=False)

        def round_body(o, carry):
            for b in range(NBUF):
                fire(o, b, first=False)
            for b in range(NBUF):
                compute(o, b, last=False)
            return carry
        lax.fori_loop(1, rounds - 1, round_body, 0, unroll=False)

        o_last = rounds - 1
        for b in range(NBUF):
            fire(o_last, b, first=False)
        for b in range(NBUF):
            compute(o_last, b, last=True)

        for b in range(NBUF):
            out_copy(base_seq + o_last * NBUF + b, b).wait()

    return emb_kernel


def kernel(x, embedding, pos_encoding):
    B, S = x.shape
    V, D = embedding.shape
    idx2d = x.astype(jnp.int32).reshape(B * S // (S // 2), S // 2)
    out = _make_kernel(B, S, D, V)(idx2d, pos_encoding, embedding)
    return out.reshape(B, S, D)
